# Initial kernel scaffold; baseline (speedup 1.0000x reference)
import jax
import jax.numpy as jnp
from jax.experimental import pallas as pl


def kernel(x, norm_weight, router_w, router_b, fc1_w, fc1_b, fc2_w, fc2_b):
    raise NotImplementedError("write your pallas kernel here")


# speedup vs baseline: 1.5547x; 1.5547x over previous
import functools

import jax
import jax.numpy as jnp
from jax.experimental import pallas as pl
from jax.experimental.pallas import tpu as pltpu

_EPS = 1e-05

_BF = 512


def _mlp_body(xe_ref, w1_ref, b1_ref, w2_ref, b2_ref, gate_ref, out_ref, acc_ref):
    e = pl.program_id(0)
    f = pl.program_id(1)
    nf = pl.num_programs(1)
    bf = w1_ref.shape[1]

    x = xe_ref[0]
    w1 = w1_ref[0]
    h = jax.lax.dot_general(
        x, w1, (((1,), (1,)), ((), ())),
        preferred_element_type=jnp.float32,
        precision=jax.lax.Precision.DEFAULT,
    )
    b1 = b1_ref[e, pl.ds(f * bf, bf)]
    h = h + b1[None, :]
    h = h * 0.5 * (1.0 + jax.lax.erf(h * 0.7071067811865476))
    w2 = w2_ref[0]
    contrib = jax.lax.dot_general(
        h, w2, (((1,), (1,)), ((), ())),
        preferred_element_type=jnp.float32,
        precision=jax.lax.Precision.DEFAULT,
    )

    @pl.when(f == 0)
    def _():
        acc_ref[...] = contrib

    @pl.when(f > 0)
    def _():
        acc_ref[...] = acc_ref[...] + contrib

    @pl.when(f == nf - 1)
    def _():
        b2 = b2_ref[e]
        g = gate_ref[e]
        out_ref[0] = (acc_ref[...] + b2[None, :]) * g[:, None]


def _expert_mlp(x_e, fc1_w, fc1_b, fc2_w, fc2_b, gate_vals):
    E, C, H = x_e.shape
    FFN = fc1_w.shape[1]
    bf = _BF if FFN % _BF == 0 else FFN
    nf = FFN // bf
    grid = (E, nf)
    return pl.pallas_call(
        _mlp_body,
        grid=grid,
        in_specs=[
            pl.BlockSpec((1, C, H), lambda e, f: (e, 0, 0)),
            pl.BlockSpec((1, bf, H), lambda e, f: (e, f, 0)),
            pl.BlockSpec((E, FFN), lambda e, f: (0, 0)),
            pl.BlockSpec((1, H, bf), lambda e, f: (e, 0, f)),
            pl.BlockSpec((E, H), lambda e, f: (0, 0)),
            pl.BlockSpec((E, C), lambda e, f: (0, 0)),
        ],
        out_specs=pl.BlockSpec((1, C, H), lambda e, f: (e, 0, 0)),
        out_shape=jax.ShapeDtypeStruct((E, C, H), jnp.float32),
        scratch_shapes=[pltpu.VMEM((C, H), jnp.float32)],
        compiler_params=pltpu.CompilerParams(
            dimension_semantics=("parallel", "arbitrary"),
        ),
    )(x_e, fc1_w, fc1_b, fc2_w, fc2_b, gate_vals)


def kernel(x, norm_weight, router_w, router_b, fc1_w, fc1_b, fc2_w, fc2_b):
    Bv, Sv, Hv = x.shape
    T = Bv * Sv
    Ev = router_w.shape[0]
    x_flat = x.reshape(T, Hv)
    ms = jnp.mean(x_flat * x_flat, axis=-1, keepdims=True)
    x_norm = x_flat * jax.lax.rsqrt(ms + _EPS) * norm_weight
    router_logits = x_norm @ router_w.T + router_b
    router_probs = jax.nn.softmax(router_logits, axis=-1)
    C = T // Ev
    gate_vals, token_idx = jax.lax.top_k(router_probs.T, C)

    x_e = x_norm[token_idx]
    y = _expert_mlp(x_e, fc1_w, fc1_b, fc2_w, fc2_b, gate_vals)
    out = jnp.zeros((T, Hv), dtype=x.dtype).at[token_idx.reshape(-1)].add(
        y.reshape(-1, Hv))
    return out.reshape(Bv, Sv, Hv)


# speedup vs baseline: 1.7246x; 1.1093x over previous
import functools

import jax
import jax.numpy as jnp
from jax.experimental import pallas as pl
from jax.experimental.pallas import tpu as pltpu

_EPS = 1e-05

_BF = 1024


def _mlp_body(xe_ref, w1_ref, b1_ref, w2_ref, b2_ref, gate_ref, out_ref, acc_ref):
    e = pl.program_id(0)
    f = pl.program_id(1)
    nf = pl.num_programs(1)
    bf = w1_ref.shape[1]

    x = xe_ref[0]
    w1 = w1_ref[0].astype(jnp.bfloat16)
    h = jax.lax.dot_general(
        x, w1, (((1,), (1,)), ((), ())),
        preferred_element_type=jnp.float32,
    )
    b1 = b1_ref[e, pl.ds(f * bf, bf)]
    h = h + b1[None, :]
    h = h * 0.5 * (1.0 + jax.lax.erf(h * 0.7071067811865476))
    h = h.astype(jnp.bfloat16)
    w2 = w2_ref[0].astype(jnp.bfloat16)
    contrib = jax.lax.dot_general(
        h, w2, (((1,), (1,)), ((), ())),
        preferred_element_type=jnp.float32,
    )

    @pl.when(f == 0)
    def _():
        acc_ref[...] = contrib

    @pl.when(f > 0)
    def _():
        acc_ref[...] = acc_ref[...] + contrib

    @pl.when(f == nf - 1)
    def _():
        b2 = b2_ref[e]
        g = gate_ref[e]
        out_ref[0] = (acc_ref[...] + b2[None, :]) * g[:, None]


def _expert_mlp(x_e, fc1_w, fc1_b, fc2_w, fc2_b, gate_vals):
    E, C, H = x_e.shape
    FFN = fc1_w.shape[1]
    bf = _BF if FFN % _BF == 0 else FFN
    nf = FFN // bf
    grid = (E, nf)
    return pl.pallas_call(
        _mlp_body,
        grid=grid,
        in_specs=[
            pl.BlockSpec((1, C, H), lambda e, f: (e, 0, 0)),
            pl.BlockSpec((1, bf, H), lambda e, f: (e, f, 0)),
            pl.BlockSpec((E, FFN), lambda e, f: (0, 0)),
            pl.BlockSpec((1, H, bf), lambda e, f: (e, 0, f)),
            pl.BlockSpec((E, H), lambda e, f: (0, 0)),
            pl.BlockSpec((E, C), lambda e, f: (0, 0)),
        ],
        out_specs=pl.BlockSpec((1, C, H), lambda e, f: (e, 0, 0)),
        out_shape=jax.ShapeDtypeStruct((E, C, H), jnp.float32),
        scratch_shapes=[pltpu.VMEM((C, H), jnp.float32)],
        compiler_params=pltpu.CompilerParams(
            dimension_semantics=("parallel", "arbitrary"),
        ),
    )(x_e, fc1_w, fc1_b, fc2_w, fc2_b, gate_vals)


def kernel(x, norm_weight, router_w, router_b, fc1_w, fc1_b, fc2_w, fc2_b):
    Bv, Sv, Hv = x.shape
    T = Bv * Sv
    Ev = router_w.shape[0]
    x_flat = x.reshape(T, Hv)
    ms = jnp.mean(x_flat * x_flat, axis=-1, keepdims=True)
    x_norm = x_flat * jax.lax.rsqrt(ms + _EPS) * norm_weight
    router_logits = x_norm @ router_w.T + router_b
    router_probs = jax.nn.softmax(router_logits, axis=-1)
    C = T // Ev
    gate_vals, token_idx = jax.lax.top_k(router_probs.T, C)

    x_e = x_norm.astype(jnp.bfloat16)[token_idx]
    y = _expert_mlp(x_e, fc1_w, fc1_b, fc2_w, fc2_b, gate_vals)
    out = jnp.zeros((T, Hv), dtype=x.dtype).at[token_idx.reshape(-1)].add(
        y.reshape(-1, Hv))
    return out.reshape(Bv, Sv, Hv)
